# baseline (speedup 1.0000x reference)
import functools

import jax
import jax.numpy as jnp
from jax.experimental import pallas as pl
from jax.experimental.pallas import tpu as pltpu
from jax.scipy.special import ndtri

B = 32
VOCAB = 1000000
SEED = 42
K_PARAM = 4
B_PARAM = 8
PRIOR_TOKENS = 4
TOP_P = 0.9
NUM_DRAFTS = 8

FILL_BLK = 32768
FILL_NBLK = (VOCAB + FILL_BLK - 1) // FILL_BLK

DRAW_BLK = 8192
DRAW_NBLK = (VOCAB + DRAW_BLK - 1) // DRAW_BLK

REC_BLK = 8192
REC_NBLK = (VOCAB + REC_BLK - 1) // REC_BLK

_U32 = jnp.uint32


def _mix32(x):
    x = x.astype(jnp.uint32)
    x = x ^ (x >> 16)
    x = x * jnp.uint32(0x7FEB352D)
    x = x ^ (x >> 15)
    x = x * jnp.uint32(0x846CA68B)
    x = x ^ (x >> 16)
    return x


def _seed_fn(prefix_row):
    h = _mix32(prefix_row.astype(jnp.uint32) * jnp.uint32(2654435761))
    folded = jnp.sum(h, dtype=jnp.uint32)
    kt = _mix32(folded ^ jnp.uint32(SEED) ^ _mix32(jnp.uint32(K_PARAM)) ^ _mix32(jnp.uint32(B_PARAM)) ^ jnp.uint32(PRIOR_TOKENS))
    return kt


def _gauss_scores(kt, cands):
    h = _mix32(jnp.uint32(SEED) ^ _mix32(kt ^ _mix32(cands.astype(jnp.uint32))))
    u = (h.astype(jnp.float32) + 0.5) / jnp.float32(4294967296.0)
    return ndtri(u)


_ROT1 = (13, 15, 26, 6)
_ROT2 = (17, 29, 16, 24)


def _tf_rounds(x0, x1, rots):
    for r in rots:
        x0 = x0 + x1
        x1 = (x1 << _U32(r)) | (x1 >> _U32(32 - r))
        x1 = x0 ^ x1
    return x0, x1


def _tf2x32(ks0, ks1, x0, x1):
    ks2 = ks0 ^ ks1 ^ _U32(0x1BD11BDA)
    x0 = x0 + ks0
    x1 = x1 + ks1
    x0, x1 = _tf_rounds(x0, x1, _ROT1)
    x0 = x0 + ks1
    x1 = x1 + ks2 + _U32(1)
    x0, x1 = _tf_rounds(x0, x1, _ROT2)
    x0 = x0 + ks2
    x1 = x1 + ks0 + _U32(2)
    x0, x1 = _tf_rounds(x0, x1, _ROT1)
    x0 = x0 + ks0
    x1 = x1 + ks1 + _U32(3)
    x0, x1 = _tf_rounds(x0, x1, _ROT2)
    x0 = x0 + ks1
    x1 = x1 + ks2 + _U32(4)
    x0, x1 = _tf_rounds(x0, x1, _ROT1)
    x0 = x0 + ks2
    x1 = x1 + ks0 + _U32(5)
    return x0, x1


def _draws_body(logp_ref, keys_ref, cutoff_ref, out_ref, run_val, run_idx):
    b = pl.program_id(0)
    v = pl.program_id(1)

    @pl.when(v == 0)
    def _init():
        run_val[...] = jnp.full((NUM_DRAFTS, 128), -jnp.inf, jnp.float32)
        run_idx[...] = jnp.full((NUM_DRAFTS, 128), 2**30, jnp.int32)

    cutoff = cutoff_ref[b]

    @pl.when(v * DRAW_BLK <= cutoff)
    def _compute():
        ks0 = keys_ref[b, 0]
        ks1 = keys_ref[b, 1]
        j = v * DRAW_BLK + jax.lax.broadcasted_iota(jnp.int32, (NUM_DRAFTS, DRAW_BLK), 1)
        d = jax.lax.broadcasted_iota(jnp.int32, (NUM_DRAFTS, DRAW_BLK), 0)
        n = (d * jnp.int32(VOCAB) + j).astype(jnp.uint32)
        x0, x1 = _tf2x32(ks0, ks1, jnp.zeros((NUM_DRAFTS, DRAW_BLK), jnp.uint32), n)
        bits = x0 ^ x1
        ubits = (bits >> _U32(9)) | _U32(0x3F800000)
        u = jax.lax.bitcast_convert_type(ubits, jnp.float32) + jnp.float32(-1.0)
        u = jnp.maximum(jnp.float32(1.17549435e-38), u)
        g = -jnp.log(-jnp.log(u))
        logp = logp_ref[0, 0, :]
        score = g + logp[None, :]
        score = jnp.where(j < VOCAB, score, -jnp.inf)
        s3 = score.reshape(NUM_DRAFTS, DRAW_BLK // 128, 128)
        j3 = j.reshape(NUM_DRAFTS, DRAW_BLK // 128, 128)
        bmax = jnp.max(s3, axis=1)
        bidx = jnp.min(jnp.where(s3 == bmax[:, None, :], j3, 2**30), axis=1)
        take = bmax > run_val[...]
        run_idx[...] = jnp.where(take, bidx, run_idx[...])
        run_val[...] = jnp.where(take, bmax, run_val[...])

    @pl.when(v == DRAW_NBLK - 1)
    def _fin():
        rv = run_val[...]
        ri = run_idx[...]
        fmax = jnp.max(rv, axis=1, keepdims=True)
        fidx = jnp.min(jnp.where(rv == fmax, ri, 2**30), axis=1, keepdims=True)
        out_ref[...] = fidx.reshape(1, 1, NUM_DRAFTS)


def _draws(logp, keys, cutoff):
    return pl.pallas_call(
        _draws_body,
        grid=(B, DRAW_NBLK),
        in_specs=[
            pl.BlockSpec((1, 1, DRAW_BLK), lambda b, v: (b, 0, v)),
            pl.BlockSpec(memory_space=pltpu.SMEM),
            pl.BlockSpec(memory_space=pltpu.SMEM),
        ],
        out_specs=pl.BlockSpec((1, 1, NUM_DRAFTS), lambda b, v: (b, 0, 0)),
        out_shape=jax.ShapeDtypeStruct((B, 1, NUM_DRAFTS), jnp.int32),
        scratch_shapes=[
            pltpu.VMEM((NUM_DRAFTS, 128), jnp.float32),
            pltpu.VMEM((NUM_DRAFTS, 128), jnp.int32),
        ],
    )(logp.reshape(B, 1, VOCAB), keys, cutoff)


def _count_body(probs_ref, vstar_ref, out_ref, acc):
    b = pl.program_id(0)
    v = pl.program_id(1)

    @pl.when(v == 0)
    def _init():
        acc[...] = jnp.zeros((NUM_DRAFTS, 128), jnp.int32)

    p = probs_ref[0, 0, :]
    j = v * REC_BLK + jax.lax.broadcasted_iota(jnp.int32, (1, REC_BLK), 1)
    vs = vstar_ref[0, 0, :]
    gt = (p[None, :] > vs[:, None]) & (j < VOCAB)
    g3 = gt.astype(jnp.int32).reshape(NUM_DRAFTS, REC_BLK // 128, 128)
    acc[...] = acc[...] + jnp.sum(g3, axis=1)

    @pl.when(v == REC_NBLK - 1)
    def _fin():
        out_ref[...] = acc[...].reshape(1, NUM_DRAFTS, 128)


def _count_gt(probs, vstar):
    partial = pl.pallas_call(
        _count_body,
        grid=(B, REC_NBLK),
        in_specs=[
            pl.BlockSpec((1, 1, REC_BLK), lambda b, v: (b, 0, v)),
            pl.BlockSpec((1, 1, NUM_DRAFTS), lambda b, v: (b, 0, 0)),
        ],
        out_specs=pl.BlockSpec((1, NUM_DRAFTS, 128), lambda b, v: (b, 0, 0)),
        out_shape=jax.ShapeDtypeStruct((B, NUM_DRAFTS, 128), jnp.int32),
        scratch_shapes=[pltpu.VMEM((NUM_DRAFTS, 128), jnp.int32)],
    )(probs.reshape(B, 1, VOCAB), vstar.reshape(B, 1, NUM_DRAFTS))
    return jnp.sum(partial, axis=-1)


def _prefix_lanes(x):
    n = x.shape[-1]
    lane = jax.lax.broadcasted_iota(jnp.int32, x.shape, len(x.shape) - 1)
    k = 1
    while k < n:
        rolled = pltpu.roll(x, k, axis=len(x.shape) - 1)
        x = x + jnp.where(lane >= k, rolled, 0)
        k *= 2
    return x


def _select_body(probs_ref, vstar_ref, rank_ref, out_ref, run_cnt):
    b = pl.program_id(0)
    v = pl.program_id(1)

    @pl.when(v == 0)
    def _init():
        for dd in range(NUM_DRAFTS):
            run_cnt[0, dd] = 0
            out_ref[b, dd] = 0

    p = probs_ref[0, 0, :]
    j = (v * REC_BLK + jax.lax.broadcasted_iota(jnp.int32, (1, REC_BLK), 1))
    vs = vstar_ref[0, 0, :]
    eq = (p[None, :] == vs[:, None]) & (j < VOCAB)
    eqi = eq.astype(jnp.int32)
    for dd in range(NUM_DRAFTS):
        rem = rank_ref[b, dd] - run_cnt[0, dd]
        cnt_d = jnp.sum(eqi[dd : dd + 1, :])

        @pl.when((rem >= 0) & (rem < cnt_d))
        def _hit(dd=dd, rem=rem):
            pref = _prefix_lanes(eqi[dd : dd + 1, :])
            tgt = jnp.min(jnp.where(eq[dd : dd + 1, :] & (pref == rem + 1), j, 2**30))
            out_ref[b, dd] = tgt

        run_cnt[0, dd] = run_cnt[0, dd] + cnt_d


def _select_tokens(probs, vstar, rank):
    return pl.pallas_call(
        _select_body,
        grid=(B, REC_NBLK),
        in_specs=[
            pl.BlockSpec((1, 1, REC_BLK), lambda b, v: (b, 0, v)),
            pl.BlockSpec((1, 1, NUM_DRAFTS), lambda b, v: (b, 0, 0)),
            pl.BlockSpec(memory_space=pltpu.SMEM),
        ],
        out_specs=pl.BlockSpec(memory_space=pltpu.SMEM),
        out_shape=jax.ShapeDtypeStruct((B, NUM_DRAFTS), jnp.int32),
        scratch_shapes=[pltpu.SMEM((1, NUM_DRAFTS), jnp.int32)],
    )(probs.reshape(B, 1, VOCAB), vstar.reshape(B, 1, NUM_DRAFTS), rank)


def _fill_body(best_ref, out_ref):
    v = pl.program_id(0)
    cols = v * FILL_BLK + jax.lax.broadcasted_iota(jnp.int32, (B, FILL_BLK), 1)
    out_ref[...] = jnp.where(cols == best_ref[...], jnp.float32(100000.0), jnp.float32(1e-05))


def _fill(best):
    return pl.pallas_call(
        _fill_body,
        grid=(FILL_NBLK,),
        in_specs=[pl.BlockSpec((B, 1), lambda v: (0, 0))],
        out_specs=pl.BlockSpec((B, FILL_BLK), lambda v: (0, v)),
        out_shape=jax.ShapeDtypeStruct((B, VOCAB), jnp.float32),
    )(best)


def _probs_row(logits_row):
    return jax.nn.softmax(logits_row, axis=-1)


def _cum_cutoff_row(sorted_row):
    cum = jnp.cumsum(sorted_row)
    cutoff = jnp.searchsorted(cum, jnp.float32(TOP_P), side='left')
    cutoff = jnp.minimum(cutoff, VOCAB - 1)
    return cutoff


def _logp_row(sorted_row, cutoff):
    mask = jnp.arange(VOCAB) <= cutoff
    kept = jnp.where(mask & jnp.isfinite(sorted_row), sorted_row, 0.0)
    logp = jnp.where(kept > 0, jnp.log(jnp.maximum(kept, 1e-37)), -jnp.inf)
    return logp


def kernel(input_ids, logits):
    probs = jax.vmap(_probs_row)(logits)
    sorted_probs = jnp.flip(jnp.sort(probs, axis=-1), axis=-1)
    cutoff = jax.vmap(_cum_cutoff_row)(sorted_probs)
    logp = jax.vmap(_logp_row)(sorted_probs, cutoff)

    keys = jax.random.split(jax.random.key(SEED), B)
    keydata = jax.random.key_data(keys).astype(jnp.uint32)

    jstar = _draws(logp, keydata, cutoff.astype(jnp.int32)).reshape(B, NUM_DRAFTS)
    vstar = jnp.take_along_axis(sorted_probs, jstar, axis=1)

    count_gt = _count_gt(probs, vstar).reshape(B, NUM_DRAFTS)
    rank = jstar - count_gt
    cands = _select_tokens(probs, vstar, rank).reshape(B, NUM_DRAFTS)

    kt = jax.vmap(_seed_fn)(input_ids)
    u = jax.vmap(_gauss_scores)(kt, cands)
    best = jnp.take_along_axis(cands, jnp.argmax(u, axis=1)[:, None], axis=1)

    return _fill(best.astype(jnp.int32))


# speedup vs baseline: 1.7786x; 1.7786x over previous
import functools

import jax
import jax.numpy as jnp
from jax.experimental import pallas as pl
from jax.experimental.pallas import tpu as pltpu
from jax.scipy.special import ndtri

B = 32
VOCAB = 1000000
SEED = 42
K_PARAM = 4
B_PARAM = 8
PRIOR_TOKENS = 4
TOP_P = 0.9
NUM_DRAFTS = 8

FILL_BLK = 32768
FILL_NBLK = (VOCAB + FILL_BLK - 1) // FILL_BLK

DRAW_BLK = 16384
DRAW_NBLK = (VOCAB + DRAW_BLK - 1) // DRAW_BLK

REC_BLK = 32768
REC_NBLK = (VOCAB + REC_BLK - 1) // REC_BLK

_U32 = jnp.uint32


def _mix32(x):
    x = x.astype(jnp.uint32)
    x = x ^ (x >> 16)
    x = x * jnp.uint32(0x7FEB352D)
    x = x ^ (x >> 15)
    x = x * jnp.uint32(0x846CA68B)
    x = x ^ (x >> 16)
    return x


def _seed_fn(prefix_row):
    h = _mix32(prefix_row.astype(jnp.uint32) * jnp.uint32(2654435761))
    folded = jnp.sum(h, dtype=jnp.uint32)
    kt = _mix32(folded ^ jnp.uint32(SEED) ^ _mix32(jnp.uint32(K_PARAM)) ^ _mix32(jnp.uint32(B_PARAM)) ^ jnp.uint32(PRIOR_TOKENS))
    return kt


def _gauss_scores(kt, cands):
    h = _mix32(jnp.uint32(SEED) ^ _mix32(kt ^ _mix32(cands.astype(jnp.uint32))))
    u = (h.astype(jnp.float32) + 0.5) / jnp.float32(4294967296.0)
    return ndtri(u)


_ROT1 = (13, 15, 26, 6)
_ROT2 = (17, 29, 16, 24)


def _tf_rounds(x0, x1, rots):
    for r in rots:
        x0 = x0 + x1
        x1 = (x1 << _U32(r)) | (x1 >> _U32(32 - r))
        x1 = x0 ^ x1
    return x0, x1


def _tf2x32(ks0, ks1, x0, x1):
    ks2 = ks0 ^ ks1 ^ _U32(0x1BD11BDA)
    x0 = x0 + ks0
    x1 = x1 + ks1
    x0, x1 = _tf_rounds(x0, x1, _ROT1)
    x0 = x0 + ks1
    x1 = x1 + ks2 + _U32(1)
    x0, x1 = _tf_rounds(x0, x1, _ROT2)
    x0 = x0 + ks2
    x1 = x1 + ks0 + _U32(2)
    x0, x1 = _tf_rounds(x0, x1, _ROT1)
    x0 = x0 + ks0
    x1 = x1 + ks1 + _U32(3)
    x0, x1 = _tf_rounds(x0, x1, _ROT2)
    x0 = x0 + ks1
    x1 = x1 + ks2 + _U32(4)
    x0, x1 = _tf_rounds(x0, x1, _ROT1)
    x0 = x0 + ks2
    x1 = x1 + ks0 + _U32(5)
    return x0, x1


def _draws_body(logp_ref, keys_ref, cutoff_ref, out_ref, run_val, run_idx):
    b = pl.program_id(0)
    v = pl.program_id(1)

    @pl.when(v == 0)
    def _init():
        run_val[...] = jnp.full((NUM_DRAFTS, 128), -jnp.inf, jnp.float32)
        run_idx[...] = jnp.full((NUM_DRAFTS, 128), 2**30, jnp.int32)

    cutoff = cutoff_ref[b]

    @pl.when(v * DRAW_BLK <= cutoff)
    def _compute():
        ks0 = keys_ref[b, 0]
        ks1 = keys_ref[b, 1]
        j = v * DRAW_BLK + jax.lax.broadcasted_iota(jnp.int32, (NUM_DRAFTS, DRAW_BLK), 1)
        d = jax.lax.broadcasted_iota(jnp.int32, (NUM_DRAFTS, DRAW_BLK), 0)
        n = (d * jnp.int32(VOCAB) + j).astype(jnp.uint32)
        x0, x1 = _tf2x32(ks0, ks1, jnp.zeros((NUM_DRAFTS, DRAW_BLK), jnp.uint32), n)
        bits = x0 ^ x1
        ubits = (bits >> _U32(9)) | _U32(0x3F800000)
        u = jax.lax.bitcast_convert_type(ubits, jnp.float32) + jnp.float32(-1.0)
        u = jnp.maximum(jnp.float32(1.17549435e-38), u)
        g = -jnp.log(-jnp.log(u))
        logp = logp_ref[0, 0, :]
        score = g + logp[None, :]
        score = jnp.where(j < VOCAB, score, -jnp.inf)
        s3 = score.reshape(NUM_DRAFTS, DRAW_BLK // 128, 128)
        j3 = j.reshape(NUM_DRAFTS, DRAW_BLK // 128, 128)
        bmax = jnp.max(s3, axis=1)
        bidx = jnp.min(jnp.where(s3 == bmax[:, None, :], j3, 2**30), axis=1)
        take = bmax > run_val[...]
        run_idx[...] = jnp.where(take, bidx, run_idx[...])
        run_val[...] = jnp.where(take, bmax, run_val[...])

    @pl.when(v == DRAW_NBLK - 1)
    def _fin():
        rv = run_val[...]
        ri = run_idx[...]
        fmax = jnp.max(rv, axis=1, keepdims=True)
        fidx = jnp.min(jnp.where(rv == fmax, ri, 2**30), axis=1, keepdims=True)
        out_ref[...] = fidx.reshape(1, 1, NUM_DRAFTS)


def _draws(logp, keys, cutoff):
    return pl.pallas_call(
        _draws_body,
        grid=(B, DRAW_NBLK),
        in_specs=[
            pl.BlockSpec((1, 1, DRAW_BLK), lambda b, v: (b, 0, v)),
            pl.BlockSpec(memory_space=pltpu.SMEM),
            pl.BlockSpec(memory_space=pltpu.SMEM),
        ],
        out_specs=pl.BlockSpec((1, 1, NUM_DRAFTS), lambda b, v: (b, 0, 0)),
        out_shape=jax.ShapeDtypeStruct((B, 1, NUM_DRAFTS), jnp.int32),
        scratch_shapes=[
            pltpu.VMEM((NUM_DRAFTS, 128), jnp.float32),
            pltpu.VMEM((NUM_DRAFTS, 128), jnp.int32),
        ],
    )(logp.reshape(B, 1, VOCAB), keys, cutoff)


def _count_body(probs_ref, vstar_ref, out_ref, acc):
    b = pl.program_id(0)
    v = pl.program_id(1)

    @pl.when(v == 0)
    def _init():
        acc[...] = jnp.zeros((NUM_DRAFTS, 128), jnp.int32)

    p = probs_ref[0, 0, :]
    j = v * REC_BLK + jax.lax.broadcasted_iota(jnp.int32, (1, REC_BLK), 1)
    vs = vstar_ref[0, 0, :]
    gt = (p[None, :] > vs[:, None]) & (j < VOCAB)
    g3 = gt.astype(jnp.int32).reshape(NUM_DRAFTS, REC_BLK // 128, 128)
    acc[...] = acc[...] + jnp.sum(g3, axis=1)

    @pl.when(v == REC_NBLK - 1)
    def _fin():
        out_ref[...] = acc[...].reshape(1, NUM_DRAFTS, 128)


def _count_gt(probs, vstar):
    partial = pl.pallas_call(
        _count_body,
        grid=(B, REC_NBLK),
        in_specs=[
            pl.BlockSpec((1, 1, REC_BLK), lambda b, v: (b, 0, v)),
            pl.BlockSpec((1, 1, NUM_DRAFTS), lambda b, v: (b, 0, 0)),
        ],
        out_specs=pl.BlockSpec((1, NUM_DRAFTS, 128), lambda b, v: (b, 0, 0)),
        out_shape=jax.ShapeDtypeStruct((B, NUM_DRAFTS, 128), jnp.int32),
        scratch_shapes=[pltpu.VMEM((NUM_DRAFTS, 128), jnp.int32)],
    )(probs.reshape(B, 1, VOCAB), vstar.reshape(B, 1, NUM_DRAFTS))
    return jnp.sum(partial, axis=-1)


def _prefix_lanes(x):
    n = x.shape[-1]
    lane = jax.lax.broadcasted_iota(jnp.int32, x.shape, len(x.shape) - 1)
    k = 1
    while k < n:
        rolled = pltpu.roll(x, k, axis=len(x.shape) - 1)
        x = x + jnp.where(lane >= k, rolled, 0)
        k *= 2
    return x


def _select_body(probs_ref, vstar_ref, rank_ref, out_ref, run_cnt):
    b = pl.program_id(0)
    v = pl.program_id(1)

    @pl.when(v == 0)
    def _init():
        for dd in range(NUM_DRAFTS):
            run_cnt[0, dd] = 0
            out_ref[b, dd] = 0

    p = probs_ref[0, 0, :]
    j = (v * REC_BLK + jax.lax.broadcasted_iota(jnp.int32, (1, REC_BLK), 1))
    vs = vstar_ref[0, 0, :]
    eq = (p[None, :] == vs[:, None]) & (j < VOCAB)
    eqi = eq.astype(jnp.int32)
    for dd in range(NUM_DRAFTS):
        rem = rank_ref[b, dd] - run_cnt[0, dd]
        cnt_d = jnp.sum(eqi[dd : dd + 1, :])

        @pl.when((rem >= 0) & (rem < cnt_d))
        def _hit(dd=dd, rem=rem):
            pref = _prefix_lanes(eqi[dd : dd + 1, :])
            tgt = jnp.min(jnp.where(eq[dd : dd + 1, :] & (pref == rem + 1), j, 2**30))
            out_ref[b, dd] = tgt

        run_cnt[0, dd] = run_cnt[0, dd] + cnt_d


def _select_tokens(probs, vstar, rank):
    return pl.pallas_call(
        _select_body,
        grid=(B, REC_NBLK),
        in_specs=[
            pl.BlockSpec((1, 1, REC_BLK), lambda b, v: (b, 0, v)),
            pl.BlockSpec((1, 1, NUM_DRAFTS), lambda b, v: (b, 0, 0)),
            pl.BlockSpec(memory_space=pltpu.SMEM),
        ],
        out_specs=pl.BlockSpec(memory_space=pltpu.SMEM),
        out_shape=jax.ShapeDtypeStruct((B, NUM_DRAFTS), jnp.int32),
        scratch_shapes=[pltpu.SMEM((1, NUM_DRAFTS), jnp.int32)],
    )(probs.reshape(B, 1, VOCAB), vstar.reshape(B, 1, NUM_DRAFTS), rank)


def _fill_body(best_ref, out_ref):
    v = pl.program_id(0)
    cols = v * FILL_BLK + jax.lax.broadcasted_iota(jnp.int32, (B, FILL_BLK), 1)
    out_ref[...] = jnp.where(cols == best_ref[...], jnp.float32(100000.0), jnp.float32(1e-05))


def _fill(best):
    return pl.pallas_call(
        _fill_body,
        grid=(FILL_NBLK,),
        in_specs=[pl.BlockSpec((B, 1), lambda v: (0, 0))],
        out_specs=pl.BlockSpec((B, FILL_BLK), lambda v: (0, v)),
        out_shape=jax.ShapeDtypeStruct((B, VOCAB), jnp.float32),
    )(best)


def _probs_row(logits_row):
    return jax.nn.softmax(logits_row, axis=-1)


def _cum_cutoff_row(sorted_row):
    cum = jnp.cumsum(sorted_row)
    cutoff = jnp.searchsorted(cum, jnp.float32(TOP_P), side='left')
    cutoff = jnp.minimum(cutoff, VOCAB - 1)
    return cutoff


def _logp_row(sorted_row, cutoff):
    mask = jnp.arange(VOCAB) <= cutoff
    kept = jnp.where(mask & jnp.isfinite(sorted_row), sorted_row, 0.0)
    logp = jnp.where(kept > 0, jnp.log(jnp.maximum(kept, 1e-37)), -jnp.inf)
    return logp


def kernel(input_ids, logits):
    probs = jax.vmap(_probs_row)(logits)
    (neg_sorted,) = jax.lax.sort((-probs,), dimension=1, is_stable=False, num_keys=1)
    sorted_probs = -neg_sorted
    cutoff = jax.vmap(_cum_cutoff_row)(sorted_probs)
    logp = jax.vmap(_logp_row)(sorted_probs, cutoff)

    keys = jax.random.split(jax.random.key(SEED), B)
    keydata = jax.random.key_data(keys).astype(jnp.uint32)

    jstar = _draws(logp, keydata, cutoff.astype(jnp.int32)).reshape(B, NUM_DRAFTS)
    vstar = jnp.take_along_axis(sorted_probs, jstar, axis=1)

    count_gt = _count_gt(probs, vstar).reshape(B, NUM_DRAFTS)
    rank = jstar - count_gt
    cands = _select_tokens(probs, vstar, rank).reshape(B, NUM_DRAFTS)

    kt = jax.vmap(_seed_fn)(input_ids)
    u = jax.vmap(_gauss_scores)(kt, cands)
    best = jnp.take_along_axis(cands, jnp.argmax(u, axis=1)[:, None], axis=1)

    return _fill(best.astype(jnp.int32))


# speedup vs baseline: 1.8638x; 1.0479x over previous
import functools

import jax
import jax.numpy as jnp
from jax.experimental import pallas as pl
from jax.experimental.pallas import tpu as pltpu
from jax.scipy.special import ndtri

B = 32
VOCAB = 1000000
SEED = 42
K_PARAM = 4
B_PARAM = 8
PRIOR_TOKENS = 4
TOP_P = 0.9
NUM_DRAFTS = 8

FILL_BLK = 32768
FILL_NBLK = (VOCAB + FILL_BLK - 1) // FILL_BLK

DRAW_BLK = 16384
DRAW_NBLK = (VOCAB + DRAW_BLK - 1) // DRAW_BLK

REC_BLK = 32768
REC_NBLK = (VOCAB + REC_BLK - 1) // REC_BLK

_U32 = jnp.uint32


def _mix32(x):
    x = x.astype(jnp.uint32)
    x = x ^ (x >> 16)
    x = x * jnp.uint32(0x7FEB352D)
    x = x ^ (x >> 15)
    x = x * jnp.uint32(0x846CA68B)
    x = x ^ (x >> 16)
    return x


def _seed_fn(prefix_row):
    h = _mix32(prefix_row.astype(jnp.uint32) * jnp.uint32(2654435761))
    folded = jnp.sum(h, dtype=jnp.uint32)
    kt = _mix32(folded ^ jnp.uint32(SEED) ^ _mix32(jnp.uint32(K_PARAM)) ^ _mix32(jnp.uint32(B_PARAM)) ^ jnp.uint32(PRIOR_TOKENS))
    return kt


def _gauss_scores(kt, cands):
    h = _mix32(jnp.uint32(SEED) ^ _mix32(kt ^ _mix32(cands.astype(jnp.uint32))))
    u = (h.astype(jnp.float32) + 0.5) / jnp.float32(4294967296.0)
    return ndtri(u)


_ROT1 = (13, 15, 26, 6)
_ROT2 = (17, 29, 16, 24)


def _tf_rounds(x0, x1, rots):
    for r in rots:
        x0 = x0 + x1
        x1 = (x1 << _U32(r)) | (x1 >> _U32(32 - r))
        x1 = x0 ^ x1
    return x0, x1


def _tf2x32(ks0, ks1, x0, x1):
    ks2 = ks0 ^ ks1 ^ _U32(0x1BD11BDA)
    x0 = x0 + ks0
    x1 = x1 + ks1
    x0, x1 = _tf_rounds(x0, x1, _ROT1)
    x0 = x0 + ks1
    x1 = x1 + ks2 + _U32(1)
    x0, x1 = _tf_rounds(x0, x1, _ROT2)
    x0 = x0 + ks2
    x1 = x1 + ks0 + _U32(2)
    x0, x1 = _tf_rounds(x0, x1, _ROT1)
    x0 = x0 + ks0
    x1 = x1 + ks1 + _U32(3)
    x0, x1 = _tf_rounds(x0, x1, _ROT2)
    x0 = x0 + ks1
    x1 = x1 + ks2 + _U32(4)
    x0, x1 = _tf_rounds(x0, x1, _ROT1)
    x0 = x0 + ks2
    x1 = x1 + ks0 + _U32(5)
    return x0, x1


def _draws_body(logp_ref, keys_ref, cutoff_ref, out_ref, run_val, run_idx):
    b = pl.program_id(0)
    v = pl.program_id(1)

    @pl.when(v == 0)
    def _init():
        run_val[...] = jnp.full((NUM_DRAFTS, 128), -jnp.inf, jnp.float32)
        run_idx[...] = jnp.full((NUM_DRAFTS, 128), 2**30, jnp.int32)

    cutoff = cutoff_ref[b]

    @pl.when(v * DRAW_BLK <= cutoff)
    def _compute():
        ks0 = keys_ref[b, 0]
        ks1 = keys_ref[b, 1]
        j = v * DRAW_BLK + jax.lax.broadcasted_iota(jnp.int32, (NUM_DRAFTS, DRAW_BLK), 1)
        d = jax.lax.broadcasted_iota(jnp.int32, (NUM_DRAFTS, DRAW_BLK), 0)
        n = (d * jnp.int32(VOCAB) + j).astype(jnp.uint32)
        x0, x1 = _tf2x32(ks0, ks1, jnp.zeros((NUM_DRAFTS, DRAW_BLK), jnp.uint32), n)
        bits = x0 ^ x1
        ubits = (bits >> _U32(9)) | _U32(0x3F800000)
        u = jax.lax.bitcast_convert_type(ubits, jnp.float32) + jnp.float32(-1.0)
        u = jnp.maximum(jnp.float32(1.17549435e-38), u)
        g = -jnp.log(-jnp.log(u))
        logp = logp_ref[0, 0, :]
        score = g + logp[None, :]
        score = jnp.where(j < VOCAB, score, -jnp.inf)
        s3 = score.reshape(NUM_DRAFTS, DRAW_BLK // 128, 128)
        j3 = j.reshape(NUM_DRAFTS, DRAW_BLK // 128, 128)
        bmax = jnp.max(s3, axis=1)
        bidx = jnp.min(jnp.where(s3 == bmax[:, None, :], j3, 2**30), axis=1)
        take = bmax > run_val[...]
        run_idx[...] = jnp.where(take, bidx, run_idx[...])
        run_val[...] = jnp.where(take, bmax, run_val[...])

    @pl.when(v == DRAW_NBLK - 1)
    def _fin():
        rv = run_val[...]
        ri = run_idx[...]
        fmax = jnp.max(rv, axis=1, keepdims=True)
        fidx = jnp.min(jnp.where(rv == fmax, ri, 2**30), axis=1, keepdims=True)
        out_ref[...] = fidx.reshape(1, 1, NUM_DRAFTS)


def _draws(logp, keys, cutoff):
    return pl.pallas_call(
        _draws_body,
        grid=(B, DRAW_NBLK),
        in_specs=[
            pl.BlockSpec((1, 1, DRAW_BLK), lambda b, v: (b, 0, v)),
            pl.BlockSpec(memory_space=pltpu.SMEM),
            pl.BlockSpec(memory_space=pltpu.SMEM),
        ],
        out_specs=pl.BlockSpec((1, 1, NUM_DRAFTS), lambda b, v: (b, 0, 0)),
        out_shape=jax.ShapeDtypeStruct((B, 1, NUM_DRAFTS), jnp.int32),
        scratch_shapes=[
            pltpu.VMEM((NUM_DRAFTS, 128), jnp.float32),
            pltpu.VMEM((NUM_DRAFTS, 128), jnp.int32),
        ],
    )(logp.reshape(B, 1, VOCAB), keys, cutoff)


def _count_body(probs_ref, vstar_ref, gt_ref, eq_ref, acc):
    b = pl.program_id(0)
    v = pl.program_id(1)

    @pl.when(v == 0)
    def _init():
        acc[...] = jnp.zeros((NUM_DRAFTS, 128), jnp.int32)

    p = probs_ref[0, 0, :]
    j = v * REC_BLK + jax.lax.broadcasted_iota(jnp.int32, (1, REC_BLK), 1)
    vs = vstar_ref[0, 0, :]
    gt = (p[None, :] > vs[:, None]) & (j < VOCAB)
    g3 = gt.astype(jnp.int32).reshape(NUM_DRAFTS, REC_BLK // 128, 128)
    acc[...] = acc[...] + jnp.sum(g3, axis=1)
    eq = (p[None, :] == vs[:, None]) & (j < VOCAB)
    e3 = eq.astype(jnp.int32).reshape(NUM_DRAFTS, REC_BLK // 128, 128)
    efold = jnp.sum(e3, axis=1)
    eq_ref[...] = jnp.sum(efold, axis=1, keepdims=True).reshape(1, 1, 1, NUM_DRAFTS)

    @pl.when(v == REC_NBLK - 1)
    def _fin():
        gt_ref[...] = acc[...].reshape(1, NUM_DRAFTS, 128)


def _count_gt(probs, vstar):
    partial, eq_blk = pl.pallas_call(
        _count_body,
        grid=(B, REC_NBLK),
        in_specs=[
            pl.BlockSpec((1, 1, REC_BLK), lambda b, v: (b, 0, v)),
            pl.BlockSpec((1, 1, NUM_DRAFTS), lambda b, v: (b, 0, 0)),
        ],
        out_specs=[
            pl.BlockSpec((1, NUM_DRAFTS, 128), lambda b, v: (b, 0, 0)),
            pl.BlockSpec((1, 1, 1, NUM_DRAFTS), lambda b, v: (b, v, 0, 0)),
        ],
        out_shape=[
            jax.ShapeDtypeStruct((B, NUM_DRAFTS, 128), jnp.int32),
            jax.ShapeDtypeStruct((B, REC_NBLK, 1, NUM_DRAFTS), jnp.int32),
        ],
        scratch_shapes=[pltpu.VMEM((NUM_DRAFTS, 128), jnp.int32)],
    )(probs.reshape(B, 1, VOCAB), vstar.reshape(B, 1, NUM_DRAFTS))
    return jnp.sum(partial, axis=-1), eq_blk.reshape(B, REC_NBLK, NUM_DRAFTS)


def _prefix_lanes(x):
    n = x.shape[-1]
    lane = jax.lax.broadcasted_iota(jnp.int32, x.shape, len(x.shape) - 1)
    k = 1
    while k < n:
        rolled = pltpu.roll(x, k, axis=len(x.shape) - 1)
        x = x + jnp.where(lane >= k, rolled, 0)
        k *= 2
    return x


def _select_body(cb_ref, probs_ref, vstar_ref, rem_ref, out_ref):
    b = pl.program_id(0)
    d = pl.program_id(1)
    cb = cb_ref[b, d]
    rem = rem_ref[b, d]
    vs = vstar_ref[b, d]
    p = probs_ref[0, 0, :]
    j = cb * REC_BLK + jax.lax.broadcasted_iota(jnp.int32, (1, REC_BLK), 1)
    eq = (p[None, :] == vs) & (j < VOCAB)
    pref = _prefix_lanes(eq.astype(jnp.int32))
    tgt = jnp.min(jnp.where(eq & (pref == rem + 1), j, 2**30))
    out_ref[b, d] = tgt


def _select_tokens(probs, vstar, cb, rem):
    return pl.pallas_call(
        _select_body,
        grid_spec=pltpu.PrefetchScalarGridSpec(
            num_scalar_prefetch=1,
            grid=(B, NUM_DRAFTS),
            in_specs=[
                pl.BlockSpec((1, 1, REC_BLK), lambda b, d, cb: (b, 0, cb[b, d])),
                pl.BlockSpec(memory_space=pltpu.SMEM),
                pl.BlockSpec(memory_space=pltpu.SMEM),
            ],
            out_specs=pl.BlockSpec(memory_space=pltpu.SMEM),
        ),
        out_shape=jax.ShapeDtypeStruct((B, NUM_DRAFTS), jnp.int32),
    )(cb, probs.reshape(B, 1, VOCAB), vstar, rem)


def _fill_body(best_ref, out_ref):
    v = pl.program_id(0)
    cols = v * FILL_BLK + jax.lax.broadcasted_iota(jnp.int32, (B, FILL_BLK), 1)
    out_ref[...] = jnp.where(cols == best_ref[...], jnp.float32(100000.0), jnp.float32(1e-05))


def _fill(best):
    return pl.pallas_call(
        _fill_body,
        grid=(FILL_NBLK,),
        in_specs=[pl.BlockSpec((B, 1), lambda v: (0, 0))],
        out_specs=pl.BlockSpec((B, FILL_BLK), lambda v: (0, v)),
        out_shape=jax.ShapeDtypeStruct((B, VOCAB), jnp.float32),
    )(best)


def _probs_row(logits_row):
    return jax.nn.softmax(logits_row, axis=-1)


def _cum_cutoff_row(sorted_row):
    cum = jnp.cumsum(sorted_row)
    cutoff = jnp.searchsorted(cum, jnp.float32(TOP_P), side='left')
    cutoff = jnp.minimum(cutoff, VOCAB - 1)
    return cutoff


def _logp_row(sorted_row, cutoff):
    mask = jnp.arange(VOCAB) <= cutoff
    kept = jnp.where(mask & jnp.isfinite(sorted_row), sorted_row, 0.0)
    logp = jnp.where(kept > 0, jnp.log(jnp.maximum(kept, 1e-37)), -jnp.inf)
    return logp


def kernel(input_ids, logits):
    probs = jax.vmap(_probs_row)(logits)
    (neg_sorted,) = jax.lax.sort((-probs,), dimension=1, is_stable=False, num_keys=1)
    sorted_probs = -neg_sorted
    cutoff = jax.vmap(_cum_cutoff_row)(sorted_probs)
    logp = jax.vmap(_logp_row)(sorted_probs, cutoff)

    keys = jax.random.split(jax.random.key(SEED), B)
    keydata = jax.random.key_data(keys).astype(jnp.uint32)

    jstar = _draws(logp, keydata, cutoff.astype(jnp.int32)).reshape(B, NUM_DRAFTS)
    vstar = jnp.take_along_axis(sorted_probs, jstar, axis=1)

    count_gt, eq_blk = _count_gt(probs, vstar)
    rank = jstar - count_gt.reshape(B, NUM_DRAFTS)
    excl = jnp.cumsum(eq_blk, axis=1) - eq_blk
    le = (excl <= rank[:, None, :]) & (eq_blk > 0)
    incl = jnp.cumsum(eq_blk, axis=1)
    has = (excl <= rank[:, None, :]) & (rank[:, None, :] < incl)
    cb = jnp.argmax(has, axis=1).astype(jnp.int32)
    rem = rank - jnp.take_along_axis(excl, cb[:, None, :], axis=1).reshape(B, NUM_DRAFTS)
    cands = _select_tokens(probs, vstar, cb, rem).reshape(B, NUM_DRAFTS)

    kt = jax.vmap(_seed_fn)(input_ids)
    u = jax.vmap(_gauss_scores)(kt, cands)
    best = jnp.take_along_axis(cands, jnp.argmax(u, axis=1)[:, None], axis=1)

    return _fill(best.astype(jnp.int32))
